# Initial kernel scaffold; baseline (speedup 1.0000x reference)
import jax
import jax.numpy as jnp
from jax.experimental import pallas as pl


def kernel(x, edge_index, pos, pos_init, edge_attr, Wm1, bm1, Wm2, bm2, Wn1, bn1, Wn2, bn2, Wc1, bc1, Wc2, bc2):
    raise NotImplementedError("write your pallas kernel here")


# speedup vs baseline: 4.7951x; 4.7951x over previous
import functools

import jax
import jax.numpy as jnp
from jax import lax
from jax.experimental import pallas as pl
from jax.experimental.pallas import tpu as pltpu
from jax.experimental.pallas import tpu_sc as plsc

N = 10000
E = 320000
D = 128
PW = 16
GW = 128
NBLK = 1000
EBLK = 2000

_mesh = plsc.VectorSubcoreMesh(core_axis_name="core", subcore_axis_name="subcore")
NC = 2
NS = 16
NW = NC * NS
CHUNKS = E // GW
NP = 10240
ROWS_PER_TILE = NP // NS

_untiled = pltpu.CompilerParams(use_tc_tiling_on_sc=False)


def _tables_body(x_ref, A_ref, B_ref, bm1_ref, ta_ref, tb_ref):
    x = x_ref[...]
    ta_ref[...] = lax.dot_general(x, A_ref[...], (((1,), (1,)), ((), ())),
                                  preferred_element_type=jnp.float32)
    tb_ref[...] = lax.dot_general(x, B_ref[...], (((1,), (1,)), ((), ())),
                                  preferred_element_type=jnp.float32) + bm1_ref[...]


def _build_tables(x, A, B, bm1):
    return pl.pallas_call(
        _tables_body,
        grid=(N // NBLK,),
        in_specs=[
            pl.BlockSpec((NBLK, D), lambda i: (i, 0)),
            pl.BlockSpec((D, D), lambda i: (0, 0)),
            pl.BlockSpec((D, D), lambda i: (0, 0)),
            pl.BlockSpec((1, D), lambda i: (0, 0)),
        ],
        out_specs=[
            pl.BlockSpec((NBLK, D), lambda i: (i, 0)),
            pl.BlockSpec((NBLK, D), lambda i: (i, 0)),
        ],
        out_shape=[
            jax.ShapeDtypeStruct((N, D), jnp.float32),
            jax.ShapeDtypeStruct((N, D), jnp.float32),
        ],
    )(x, A, B, bm1)


def _gather_main(ta, tb, src_row, dst_row):
    @functools.partial(
        pl.kernel,
        out_type=(
            jax.ShapeDtypeStruct((E, D), jnp.float32),
            jax.ShapeDtypeStruct((E, D), jnp.float32),
        ),
        mesh=_mesh,
    )
    def gather_kernel(ta_hbm, tb_hbm, si_hbm, di_hbm, ga_hbm, gb_hbm):
        def body(s_vmem, d_vmem, oa_vmem, ob_vmem):
            pltpu.sync_copy(ta_hbm.at[s_vmem.at[0]], oa_vmem)
            pltpu.sync_copy(tb_hbm.at[d_vmem.at[0]], ob_vmem)

        pltpu.emit_pipeline(
            body,
            grid=(CHUNKS,),
            in_specs=[
                pl.BlockSpec((1, GW), lambda i: (0, i)),
                pl.BlockSpec((1, GW), lambda i: (0, i)),
            ],
            out_specs=[
                pl.BlockSpec((GW, D), lambda i: (i, 0)),
                pl.BlockSpec((GW, D), lambda i: (i, 0)),
            ],
            core_axis_name=("core", "subcore"),
            dimension_semantics=(pltpu.PARALLEL,),
        )(si_hbm, di_hbm, ga_hbm, gb_hbm)

    return gather_kernel(ta, tb, src_row, dst_row)


def _gather_pos(P, PN, src_row, dst_row):
    @functools.partial(
        pl.kernel,
        out_type=(
            jax.ShapeDtypeStruct((E, PW), jnp.float32),
            jax.ShapeDtypeStruct((E, PW), jnp.float32),
        ),
        mesh=_mesh,
        compiler_params=_untiled,
    )
    def gather_kernel(p_hbm, pn_hbm, si_hbm, di_hbm, gp_hbm, gq_hbm):
        def body(s_vmem, d_vmem, op_vmem, oq_vmem):
            pltpu.sync_copy(p_hbm.at[s_vmem.at[0]], op_vmem)
            pltpu.sync_copy(pn_hbm.at[d_vmem.at[0]], oq_vmem)

        pltpu.emit_pipeline(
            body,
            grid=(CHUNKS,),
            in_specs=[
                pl.BlockSpec((1, GW), lambda i: (0, i)),
                pl.BlockSpec((1, GW), lambda i: (0, i)),
            ],
            out_specs=[
                pl.BlockSpec((GW, PW), lambda i: (i, 0)),
                pl.BlockSpec((GW, PW), lambda i: (i, 0)),
            ],
            core_axis_name=("core", "subcore"),
            dimension_semantics=(pltpu.PARALLEL,),
        )(si_hbm, di_hbm, gp_hbm, gq_hbm)

    return gather_kernel(P, PN, src_row, dst_row)


def _edge_body(ga_ref, gb_ref, gp_ref, gq_ref, ea_ref, WaeT_ref, wd_ref,
               WqT_ref, bq_ref, Wc2T_ref, bc2_ref, r_ref, u_ref):
    xsum = ga_ref[...] + gb_ref[...]
    pp = gp_ref[...] + gq_ref[...]
    dp = pp[:, 0:3]
    dq = pp[:, 3:6]
    d_sq = jnp.sum(dp * dp, axis=1, keepdims=True)
    d2 = jnp.sum(dq * dq, axis=1, keepdims=True)
    rinv = lax.rsqrt(d2)
    pre = xsum + d_sq * wd_ref[...] + lax.dot_general(
        ea_ref[...], WaeT_ref[...], (((1,), (0,)), ((), ())),
        preferred_element_type=jnp.float32)
    r = jnp.maximum(pre, 0.0)
    t = jnp.maximum(
        lax.dot_general(r, WqT_ref[...], (((1,), (0,)), ((), ())),
                        preferred_element_type=jnp.float32) + bq_ref[...], 0.0)
    coord = lax.dot_general(t, Wc2T_ref[...], (((1,), (0,)), ((), ())),
                            preferred_element_type=jnp.float32) + bc2_ref[...]
    pij = dq * coord * rinv
    ones = jnp.ones((r.shape[0], 1), jnp.float32)
    pad = jnp.zeros((r.shape[0], PW - 4), jnp.float32)
    r_ref[...] = r
    u_ref[...] = jnp.concatenate([pij, ones, pad], axis=1)


def _edge_compute(ga, gb, gp, gq, ea, WaeT, wd, WqT, bq, Wc2T, bc2):
    return pl.pallas_call(
        _edge_body,
        grid=(E // EBLK,),
        in_specs=[
            pl.BlockSpec((EBLK, D), lambda i: (i, 0)),
            pl.BlockSpec((EBLK, D), lambda i: (i, 0)),
            pl.BlockSpec((EBLK, PW), lambda i: (i, 0)),
            pl.BlockSpec((EBLK, PW), lambda i: (i, 0)),
            pl.BlockSpec((EBLK, 4), lambda i: (i, 0)),
            pl.BlockSpec((4, D), lambda i: (0, 0)),
            pl.BlockSpec((1, D), lambda i: (0, 0)),
            pl.BlockSpec((D, D), lambda i: (0, 0)),
            pl.BlockSpec((1, D), lambda i: (0, 0)),
            pl.BlockSpec((D, 3), lambda i: (0, 0)),
            pl.BlockSpec((1, 3), lambda i: (0, 0)),
        ],
        out_specs=[
            pl.BlockSpec((EBLK, D), lambda i: (i, 0)),
            pl.BlockSpec((EBLK, PW), lambda i: (i, 0)),
        ],
        out_shape=[
            jax.ShapeDtypeStruct((E, D), jnp.float32),
            jax.ShapeDtypeStruct((E, PW), jnp.float32),
        ],
    )(ga, gb, gp, gq, ea, WaeT, wd, WqT, bq, Wc2T, bc2)


def _scatter_main(v, dst2d, zrows):
    @functools.partial(
        pl.kernel,
        out_type=jax.ShapeDtypeStruct((NC, NP, D), jnp.float32),
        mesh=_mesh,
        scratch_types=[
            pltpu.VMEM((GW, D), jnp.float32),
            pltpu.VMEM((GW,), jnp.int32),
            pltpu.VMEM_SHARED((NP, D), jnp.float32),
        ],
    )
    def scatter_kernel(v_hbm, di_hbm, z_hbm, o_hbm, rv, iv, acc_sh):
        cid = lax.axis_index("core")
        sid = lax.axis_index("subcore")
        wid = sid * NC + cid
        pltpu.sync_copy(z_hbm, acc_sh.at[pl.ds(sid * ROWS_PER_TILE, ROWS_PER_TILE)])
        plsc.subcore_barrier()

        @pl.loop(wid, CHUNKS, step=NW)
        def _(c):
            pltpu.sync_copy(di_hbm.at[c], iv)
            pltpu.sync_copy(v_hbm.at[pl.ds(c * GW, GW)], rv)
            pltpu.sync_copy(rv, acc_sh.at[iv], add=True)

        plsc.subcore_barrier()
        pltpu.sync_copy(
            acc_sh.at[pl.ds(sid * ROWS_PER_TILE, ROWS_PER_TILE)],
            o_hbm.at[cid, pl.ds(sid * ROWS_PER_TILE, ROWS_PER_TILE)],
        )

    return scatter_kernel(v, dst2d, zrows)


def _scatter_pos(u, dst2d, zrows):
    @functools.partial(
        pl.kernel,
        out_type=jax.ShapeDtypeStruct((NC, NP, PW), jnp.float32),
        mesh=_mesh,
        compiler_params=_untiled,
        scratch_types=[
            pltpu.VMEM((GW, PW), jnp.float32),
            pltpu.VMEM((GW,), jnp.int32),
            pltpu.VMEM_SHARED((NP, PW), jnp.float32),
        ],
    )
    def scatter_kernel(u_hbm, di_hbm, z_hbm, o_hbm, uv, iv, acc_sh):
        cid = lax.axis_index("core")
        sid = lax.axis_index("subcore")
        wid = sid * NC + cid
        pltpu.sync_copy(z_hbm, acc_sh.at[pl.ds(sid * ROWS_PER_TILE, ROWS_PER_TILE)])
        plsc.subcore_barrier()

        @pl.loop(wid, CHUNKS, step=NW)
        def _(c):
            pltpu.sync_copy(di_hbm.at[c], iv)
            pltpu.sync_copy(u_hbm.at[pl.ds(c * GW, GW)], uv)
            pltpu.sync_copy(uv, acc_sh.at[iv], add=True)

        plsc.subcore_barrier()
        pltpu.sync_copy(
            acc_sh.at[pl.ds(sid * ROWS_PER_TILE, ROWS_PER_TILE)],
            o_hbm.at[cid, pl.ds(sid * ROWS_PER_TILE, ROWS_PER_TILE)],
        )

    return scatter_kernel(u, dst2d, zrows)


def _node_body(p0_ref, p1_ref, u0_ref, u1_ref, x_ref, Wm2T_ref, bm2_ref,
               Wn1aT_ref, Wn1bT_ref, bn1_ref, Wn2T_ref, bn2_ref, h_ref, pu_ref):
    S = p0_ref[...] + p1_ref[...]
    su = u0_ref[...] + u1_ref[...]
    pupd = su[:, 0:3]
    cnt = su[:, 3:4]
    msum = lax.dot_general(S, Wm2T_ref[...], (((1,), (0,)), ((), ())),
                           preferred_element_type=jnp.float32) + cnt * bm2_ref[...]
    m_i = msum / jnp.maximum(cnt, 1.0)
    h1 = jnp.maximum(
        lax.dot_general(x_ref[...], Wn1aT_ref[...], (((1,), (0,)), ((), ())),
                        preferred_element_type=jnp.float32)
        + lax.dot_general(m_i, Wn1bT_ref[...], (((1,), (0,)), ((), ())),
                          preferred_element_type=jnp.float32)
        + bn1_ref[...], 0.0)
    h_ref[...] = lax.dot_general(h1, Wn2T_ref[...], (((1,), (0,)), ((), ())),
                                 preferred_element_type=jnp.float32) + bn2_ref[...]
    pu_ref[...] = pupd


def _node_mlp(p0, p1, u0, u1, x, Wm2T, bm2, Wn1aT, Wn1bT, bn1, Wn2T, bn2):
    return pl.pallas_call(
        _node_body,
        grid=(N // NBLK,),
        in_specs=[
            pl.BlockSpec((NBLK, D), lambda i: (i, 0)),
            pl.BlockSpec((NBLK, D), lambda i: (i, 0)),
            pl.BlockSpec((NBLK, PW), lambda i: (i, 0)),
            pl.BlockSpec((NBLK, PW), lambda i: (i, 0)),
            pl.BlockSpec((NBLK, D), lambda i: (i, 0)),
            pl.BlockSpec((D, D), lambda i: (0, 0)),
            pl.BlockSpec((1, D), lambda i: (0, 0)),
            pl.BlockSpec((D, D), lambda i: (0, 0)),
            pl.BlockSpec((D, D), lambda i: (0, 0)),
            pl.BlockSpec((1, D), lambda i: (0, 0)),
            pl.BlockSpec((D, D), lambda i: (0, 0)),
            pl.BlockSpec((1, D), lambda i: (0, 0)),
        ],
        out_specs=[
            pl.BlockSpec((NBLK, D), lambda i: (i, 0)),
            pl.BlockSpec((NBLK, 3), lambda i: (i, 0)),
        ],
        out_shape=[
            jax.ShapeDtypeStruct((N, D), jnp.float32),
            jax.ShapeDtypeStruct((N, 3), jnp.float32),
        ],
    )(p0, p1, u0, u1, x, Wm2T, bm2, Wn1aT, Wn1bT, bn1, Wn2T, bn2)


def kernel(x, edge_index, pos, pos_init, edge_attr, Wm1, bm1, Wm2, bm2,
           Wn1, bn1, Wn2, bn2, Wc1, bc1, Wc2, bc2):
    src = edge_index[0].astype(jnp.int32)
    dst = edge_index[1].astype(jnp.int32)

    A = Wm1[:, :D]
    B = Wm1[:, D:2 * D]
    wd = Wm1[:, 2 * D].reshape(1, D)
    WaeT = Wm1[:, 2 * D + 1:].T
    Wq = jnp.dot(Wc1, Wm2, precision="highest")
    bq = (jnp.dot(Wc1, bm2, precision="highest") + bc1).reshape(1, D)

    P = jnp.concatenate([pos, pos_init, jnp.zeros((N, PW - 6), jnp.float32)], axis=1)

    ta, tb = _build_tables(x, A, B, bm1.reshape(1, D))
    src_row = src.reshape(1, E)
    dst_row = dst.reshape(1, E)
    ga, gb = _gather_main(ta, tb, src_row, dst_row)
    gp, gq = _gather_pos(P, -P, src_row, dst_row)
    r, u = _edge_compute(ga, gb, gp, gq, edge_attr, WaeT, wd, Wq.T, bq,
                         Wc2.T, bc2.reshape(1, 3))
    dst2d = dst.reshape(CHUNKS, GW)
    partials = _scatter_main(r, dst2d, jnp.zeros((ROWS_PER_TILE, D), jnp.float32))
    upart = _scatter_pos(u, dst2d, jnp.zeros((ROWS_PER_TILE, PW), jnp.float32))
    h, pos_upd = _node_mlp(partials[0], partials[1], upart[0], upart[1], x,
                           Wm2.T, bm2.reshape(1, D), Wn1[:, :D].T, Wn1[:, D:].T,
                           bn1.reshape(1, D), Wn2.T, bn2.reshape(1, D))
    return (h, pos_upd)


# speedup vs baseline: 5.1927x; 1.0829x over previous
import functools

import jax
import jax.numpy as jnp
from jax import lax
from jax.experimental import pallas as pl
from jax.experimental.pallas import tpu as pltpu
from jax.experimental.pallas import tpu_sc as plsc

N = 10000
E = 320000
D = 128
PW = 16
GW = 128
NBLK = 1000
EBLK = 2000

_mesh = plsc.VectorSubcoreMesh(core_axis_name="core", subcore_axis_name="subcore")
NC = 2
NS = 16
NW = NC * NS
CHUNKS = E // GW
NP = 10240
ROWS_PER_TILE = NP // NS

_untiled = pltpu.CompilerParams(use_tc_tiling_on_sc=False)


def _tables_body(x_ref, A_ref, B_ref, bm1_ref, ta_ref, tb_ref):
    x = x_ref[...]
    ta_ref[...] = lax.dot_general(x, A_ref[...], (((1,), (1,)), ((), ())),
                                  preferred_element_type=jnp.float32)
    tb_ref[...] = lax.dot_general(x, B_ref[...], (((1,), (1,)), ((), ())),
                                  preferred_element_type=jnp.float32) + bm1_ref[...]


def _build_tables(x, A, B, bm1):
    return pl.pallas_call(
        _tables_body,
        grid=(N // NBLK,),
        in_specs=[
            pl.BlockSpec((NBLK, D), lambda i: (i, 0)),
            pl.BlockSpec((D, D), lambda i: (0, 0)),
            pl.BlockSpec((D, D), lambda i: (0, 0)),
            pl.BlockSpec((1, D), lambda i: (0, 0)),
        ],
        out_specs=[
            pl.BlockSpec((NBLK, D), lambda i: (i, 0)),
            pl.BlockSpec((NBLK, D), lambda i: (i, 0)),
        ],
        out_shape=[
            jax.ShapeDtypeStruct((N, D), jnp.float32),
            jax.ShapeDtypeStruct((N, D), jnp.float32),
        ],
    )(x, A, B, bm1)


def _gather_main(ta, tb, src_row, dst_row):
    @functools.partial(
        pl.kernel,
        out_type=(
            jax.ShapeDtypeStruct((E, D), jnp.float32),
            jax.ShapeDtypeStruct((E, D), jnp.float32),
        ),
        mesh=_mesh,
    )
    def gather_kernel(ta_hbm, tb_hbm, si_hbm, di_hbm, ga_hbm, gb_hbm):
        def body(s_vmem, d_vmem, oa_vmem, ob_vmem):
            pltpu.sync_copy(ta_hbm.at[s_vmem.at[0]], oa_vmem)
            pltpu.sync_copy(tb_hbm.at[d_vmem.at[0]], ob_vmem)

        pltpu.emit_pipeline(
            body,
            grid=(CHUNKS,),
            in_specs=[
                pl.BlockSpec((1, GW), lambda i: (0, i)),
                pl.BlockSpec((1, GW), lambda i: (0, i)),
            ],
            out_specs=[
                pl.BlockSpec((GW, D), lambda i: (i, 0)),
                pl.BlockSpec((GW, D), lambda i: (i, 0)),
            ],
            core_axis_name=("core", "subcore"),
            dimension_semantics=(pltpu.PARALLEL,),
        )(si_hbm, di_hbm, ga_hbm, gb_hbm)

    return gather_kernel(ta, tb, src_row, dst_row)


KGP = 10


def _gather_pos(P, PN, src_row, dst_row):
    @functools.partial(
        pl.kernel,
        out_type=(
            jax.ShapeDtypeStruct((E, PW), jnp.float32),
            jax.ShapeDtypeStruct((E, PW), jnp.float32),
        ),
        mesh=_mesh,
        compiler_params=_untiled,
    )
    def gather_kernel(p_hbm, pn_hbm, si_hbm, di_hbm, gp_hbm, gq_hbm):
        def body(s_vmem, d_vmem, op_vmem, oq_vmem):
            for k in range(KGP):
                sl = pl.ds(k * GW, GW)
                pltpu.sync_copy(p_hbm.at[s_vmem.at[0, sl]], op_vmem.at[sl])
                pltpu.sync_copy(pn_hbm.at[d_vmem.at[0, sl]], oq_vmem.at[sl])

        pltpu.emit_pipeline(
            body,
            grid=(CHUNKS // KGP,),
            in_specs=[
                pl.BlockSpec((1, KGP * GW), lambda i: (0, i)),
                pl.BlockSpec((1, KGP * GW), lambda i: (0, i)),
            ],
            out_specs=[
                pl.BlockSpec((KGP * GW, PW), lambda i: (i, 0)),
                pl.BlockSpec((KGP * GW, PW), lambda i: (i, 0)),
            ],
            core_axis_name=("core", "subcore"),
            dimension_semantics=(pltpu.PARALLEL,),
        )(si_hbm, di_hbm, gp_hbm, gq_hbm)

    return gather_kernel(P, PN, src_row, dst_row)


def _edge_body(ga_ref, gb_ref, gp_ref, gq_ref, ea_ref, WaeT_ref, wd_ref,
               WqT_ref, bq_ref, Wc2T_ref, bc2_ref, r_ref, u_ref):
    xsum = ga_ref[...] + gb_ref[...]
    pp = gp_ref[...] + gq_ref[...]
    dp = pp[:, 0:3]
    dq = pp[:, 3:6]
    d_sq = jnp.sum(dp * dp, axis=1, keepdims=True)
    d2 = jnp.sum(dq * dq, axis=1, keepdims=True)
    rinv = lax.rsqrt(d2)
    pre = xsum + d_sq * wd_ref[...] + lax.dot_general(
        ea_ref[...], WaeT_ref[...], (((1,), (0,)), ((), ())),
        preferred_element_type=jnp.float32)
    r = jnp.maximum(pre, 0.0)
    t = jnp.maximum(
        lax.dot_general(r, WqT_ref[...], (((1,), (0,)), ((), ())),
                        preferred_element_type=jnp.float32) + bq_ref[...], 0.0)
    coord = lax.dot_general(t, Wc2T_ref[...], (((1,), (0,)), ((), ())),
                            preferred_element_type=jnp.float32) + bc2_ref[...]
    pij = dq * coord * rinv
    ones = jnp.ones((r.shape[0], 1), jnp.float32)
    pad = jnp.zeros((r.shape[0], PW - 4), jnp.float32)
    r_ref[...] = r
    u_ref[...] = jnp.concatenate([pij, ones, pad], axis=1)


def _edge_compute(ga, gb, gp, gq, ea, WaeT, wd, WqT, bq, Wc2T, bc2):
    return pl.pallas_call(
        _edge_body,
        grid=(E // EBLK,),
        in_specs=[
            pl.BlockSpec((EBLK, D), lambda i: (i, 0)),
            pl.BlockSpec((EBLK, D), lambda i: (i, 0)),
            pl.BlockSpec((EBLK, PW), lambda i: (i, 0)),
            pl.BlockSpec((EBLK, PW), lambda i: (i, 0)),
            pl.BlockSpec((EBLK, 4), lambda i: (i, 0)),
            pl.BlockSpec((4, D), lambda i: (0, 0)),
            pl.BlockSpec((1, D), lambda i: (0, 0)),
            pl.BlockSpec((D, D), lambda i: (0, 0)),
            pl.BlockSpec((1, D), lambda i: (0, 0)),
            pl.BlockSpec((D, 3), lambda i: (0, 0)),
            pl.BlockSpec((1, 3), lambda i: (0, 0)),
        ],
        out_specs=[
            pl.BlockSpec((EBLK, D), lambda i: (i, 0)),
            pl.BlockSpec((EBLK, PW), lambda i: (i, 0)),
        ],
        out_shape=[
            jax.ShapeDtypeStruct((E, D), jnp.float32),
            jax.ShapeDtypeStruct((E, PW), jnp.float32),
        ],
    )(ga, gb, gp, gq, ea, WaeT, wd, WqT, bq, Wc2T, bc2)


KS1 = 2
KS2 = 10


def _scatter_main(v, dst2d, zrows):
    @functools.partial(
        pl.kernel,
        out_type=jax.ShapeDtypeStruct((NC, NP, D), jnp.float32),
        mesh=_mesh,
        scratch_types=[
            pltpu.VMEM((KS1 * GW, D), jnp.float32),
            pltpu.VMEM((KS1, GW), jnp.int32),
            pltpu.VMEM_SHARED((NP, D), jnp.float32),
        ],
    )
    def scatter_kernel(v_hbm, di_hbm, z_hbm, o_hbm, rv, iv, acc_sh):
        cid = lax.axis_index("core")
        sid = lax.axis_index("subcore")
        wid = sid * NC + cid
        pltpu.sync_copy(z_hbm, acc_sh.at[pl.ds(sid * ROWS_PER_TILE, ROWS_PER_TILE)])
        plsc.subcore_barrier()

        @pl.loop(wid, CHUNKS // KS1, step=NW)
        def _(c):
            pltpu.sync_copy(di_hbm.at[pl.ds(c * KS1, KS1)], iv)
            pltpu.sync_copy(v_hbm.at[pl.ds(c * (KS1 * GW), KS1 * GW)], rv)
            for k in range(KS1):
                pltpu.sync_copy(rv.at[pl.ds(k * GW, GW)], acc_sh.at[iv.at[k]],
                                add=True)

        plsc.subcore_barrier()
        pltpu.sync_copy(
            acc_sh.at[pl.ds(sid * ROWS_PER_TILE, ROWS_PER_TILE)],
            o_hbm.at[cid, pl.ds(sid * ROWS_PER_TILE, ROWS_PER_TILE)],
        )

    return scatter_kernel(v, dst2d, zrows)


def _scatter_pos(u, dst2d, zrows):
    @functools.partial(
        pl.kernel,
        out_type=jax.ShapeDtypeStruct((NC, NP, PW), jnp.float32),
        mesh=_mesh,
        compiler_params=_untiled,
        scratch_types=[
            pltpu.VMEM((KS2 * GW, PW), jnp.float32),
            pltpu.VMEM((KS2, GW), jnp.int32),
            pltpu.VMEM_SHARED((NP, PW), jnp.float32),
        ],
    )
    def scatter_kernel(u_hbm, di_hbm, z_hbm, o_hbm, uv, iv, acc_sh):
        cid = lax.axis_index("core")
        sid = lax.axis_index("subcore")
        wid = sid * NC + cid
        pltpu.sync_copy(z_hbm, acc_sh.at[pl.ds(sid * ROWS_PER_TILE, ROWS_PER_TILE)])
        plsc.subcore_barrier()

        @pl.loop(wid, CHUNKS // KS2, step=NW)
        def _(c):
            pltpu.sync_copy(di_hbm.at[pl.ds(c * KS2, KS2)], iv)
            pltpu.sync_copy(u_hbm.at[pl.ds(c * (KS2 * GW), KS2 * GW)], uv)
            for k in range(KS2):
                pltpu.sync_copy(uv.at[pl.ds(k * GW, GW)], acc_sh.at[iv.at[k]],
                                add=True)

        plsc.subcore_barrier()
        pltpu.sync_copy(
            acc_sh.at[pl.ds(sid * ROWS_PER_TILE, ROWS_PER_TILE)],
            o_hbm.at[cid, pl.ds(sid * ROWS_PER_TILE, ROWS_PER_TILE)],
        )

    return scatter_kernel(u, dst2d, zrows)


def _node_body(p0_ref, p1_ref, u0_ref, u1_ref, x_ref, Wm2T_ref, bm2_ref,
               Wn1aT_ref, Wn1bT_ref, bn1_ref, Wn2T_ref, bn2_ref, h_ref, pu_ref):
    S = p0_ref[...] + p1_ref[...]
    su = u0_ref[...] + u1_ref[...]
    pupd = su[:, 0:3]
    cnt = su[:, 3:4]
    msum = lax.dot_general(S, Wm2T_ref[...], (((1,), (0,)), ((), ())),
                           preferred_element_type=jnp.float32) + cnt * bm2_ref[...]
    m_i = msum / jnp.maximum(cnt, 1.0)
    h1 = jnp.maximum(
        lax.dot_general(x_ref[...], Wn1aT_ref[...], (((1,), (0,)), ((), ())),
                        preferred_element_type=jnp.float32)
        + lax.dot_general(m_i, Wn1bT_ref[...], (((1,), (0,)), ((), ())),
                          preferred_element_type=jnp.float32)
        + bn1_ref[...], 0.0)
    h_ref[...] = lax.dot_general(h1, Wn2T_ref[...], (((1,), (0,)), ((), ())),
                                 preferred_element_type=jnp.float32) + bn2_ref[...]
    pu_ref[...] = pupd


def _node_mlp(p0, p1, u0, u1, x, Wm2T, bm2, Wn1aT, Wn1bT, bn1, Wn2T, bn2):
    return pl.pallas_call(
        _node_body,
        grid=(N // NBLK,),
        in_specs=[
            pl.BlockSpec((NBLK, D), lambda i: (i, 0)),
            pl.BlockSpec((NBLK, D), lambda i: (i, 0)),
            pl.BlockSpec((NBLK, PW), lambda i: (i, 0)),
            pl.BlockSpec((NBLK, PW), lambda i: (i, 0)),
            pl.BlockSpec((NBLK, D), lambda i: (i, 0)),
            pl.BlockSpec((D, D), lambda i: (0, 0)),
            pl.BlockSpec((1, D), lambda i: (0, 0)),
            pl.BlockSpec((D, D), lambda i: (0, 0)),
            pl.BlockSpec((D, D), lambda i: (0, 0)),
            pl.BlockSpec((1, D), lambda i: (0, 0)),
            pl.BlockSpec((D, D), lambda i: (0, 0)),
            pl.BlockSpec((1, D), lambda i: (0, 0)),
        ],
        out_specs=[
            pl.BlockSpec((NBLK, D), lambda i: (i, 0)),
            pl.BlockSpec((NBLK, 3), lambda i: (i, 0)),
        ],
        out_shape=[
            jax.ShapeDtypeStruct((N, D), jnp.float32),
            jax.ShapeDtypeStruct((N, 3), jnp.float32),
        ],
    )(p0, p1, u0, u1, x, Wm2T, bm2, Wn1aT, Wn1bT, bn1, Wn2T, bn2)


def kernel(x, edge_index, pos, pos_init, edge_attr, Wm1, bm1, Wm2, bm2,
           Wn1, bn1, Wn2, bn2, Wc1, bc1, Wc2, bc2):
    src = edge_index[0].astype(jnp.int32)
    dst = edge_index[1].astype(jnp.int32)

    A = Wm1[:, :D]
    B = Wm1[:, D:2 * D]
    wd = Wm1[:, 2 * D].reshape(1, D)
    WaeT = Wm1[:, 2 * D + 1:].T
    Wq = jnp.dot(Wc1, Wm2, precision="highest")
    bq = (jnp.dot(Wc1, bm2, precision="highest") + bc1).reshape(1, D)

    P = jnp.concatenate([pos, pos_init, jnp.zeros((N, PW - 6), jnp.float32)], axis=1)

    ta, tb = _build_tables(x, A, B, bm1.reshape(1, D))
    src_row = src.reshape(1, E)
    dst_row = dst.reshape(1, E)
    ga, gb = _gather_main(ta, tb, src_row, dst_row)
    gp, gq = _gather_pos(P, -P, src_row, dst_row)
    r, u = _edge_compute(ga, gb, gp, gq, edge_attr, WaeT, wd, Wq.T, bq,
                         Wc2.T, bc2.reshape(1, 3))
    dst2d = dst.reshape(CHUNKS, GW)
    partials = _scatter_main(r, dst2d, jnp.zeros((ROWS_PER_TILE, D), jnp.float32))
    upart = _scatter_pos(u, dst2d, jnp.zeros((ROWS_PER_TILE, PW), jnp.float32))
    h, pos_upd = _node_mlp(partials[0], partials[1], upart[0], upart[1], x,
                           Wm2.T, bm2.reshape(1, D), Wn1[:, :D].T, Wn1[:, D:].T,
                           bn1.reshape(1, D), Wn2.T, bn2.reshape(1, D))
    return (h, pos_upd)


# speedup vs baseline: 5.3783x; 1.0357x over previous
import functools

import jax
import jax.numpy as jnp
from jax import lax
from jax.experimental import pallas as pl
from jax.experimental.pallas import tpu as pltpu
from jax.experimental.pallas import tpu_sc as plsc

N = 10000
E = 320000
D = 128
PW = 16
GW = 128
NBLK = 1000
EBLK = 2000

_mesh = plsc.VectorSubcoreMesh(core_axis_name="core", subcore_axis_name="subcore")
NC = 2
NS = 16
NW = NC * NS
CHUNKS = E // GW
NP = 10240
ROWS_PER_TILE = NP // NS

_untiled = pltpu.CompilerParams(use_tc_tiling_on_sc=False)


def _tables_body(x_ref, A_ref, B_ref, bm1_ref, ta_ref, tb_ref):
    x = x_ref[...]
    ta_ref[...] = lax.dot_general(x, A_ref[...], (((1,), (1,)), ((), ())),
                                  preferred_element_type=jnp.float32)
    tb_ref[...] = lax.dot_general(x, B_ref[...], (((1,), (1,)), ((), ())),
                                  preferred_element_type=jnp.float32) + bm1_ref[...]


def _build_tables(x, A, B, bm1):
    return pl.pallas_call(
        _tables_body,
        grid=(N // NBLK,),
        in_specs=[
            pl.BlockSpec((NBLK, D), lambda i: (i, 0)),
            pl.BlockSpec((D, D), lambda i: (0, 0)),
            pl.BlockSpec((D, D), lambda i: (0, 0)),
            pl.BlockSpec((1, D), lambda i: (0, 0)),
        ],
        out_specs=[
            pl.BlockSpec((NBLK, D), lambda i: (i, 0)),
            pl.BlockSpec((NBLK, D), lambda i: (i, 0)),
        ],
        out_shape=[
            jax.ShapeDtypeStruct((N, D), jnp.float32),
            jax.ShapeDtypeStruct((N, D), jnp.float32),
        ],
    )(x, A, B, bm1)


def _gather_main(ta, tb, src_row, dst_row):
    @functools.partial(
        pl.kernel,
        out_type=(
            jax.ShapeDtypeStruct((E, D), jnp.float32),
            jax.ShapeDtypeStruct((E, D), jnp.float32),
        ),
        mesh=_mesh,
        scratch_types=[pltpu.SemaphoreType.DMA],
    )
    def gather_kernel(ta_hbm, tb_hbm, si_hbm, di_hbm, ga_hbm, gb_hbm, sem):
        def body(s_vmem, d_vmem, oa_vmem, ob_vmem):
            ca = pltpu.async_copy(ta_hbm.at[s_vmem.at[0]], oa_vmem, sem)
            cb = pltpu.async_copy(tb_hbm.at[d_vmem.at[0]], ob_vmem, sem)
            ca.wait()
            cb.wait()

        pltpu.emit_pipeline(
            body,
            grid=(CHUNKS,),
            in_specs=[
                pl.BlockSpec((1, GW), lambda i: (0, i)),
                pl.BlockSpec((1, GW), lambda i: (0, i)),
            ],
            out_specs=[
                pl.BlockSpec((GW, D), lambda i: (i, 0)),
                pl.BlockSpec((GW, D), lambda i: (i, 0)),
            ],
            core_axis_name=("core", "subcore"),
            dimension_semantics=(pltpu.PARALLEL,),
        )(si_hbm, di_hbm, ga_hbm, gb_hbm)

    return gather_kernel(ta, tb, src_row, dst_row)


KGP = 10


def _gather_pos(P, PN, src_row, dst_row):
    @functools.partial(
        pl.kernel,
        out_type=(
            jax.ShapeDtypeStruct((E, PW), jnp.float32),
            jax.ShapeDtypeStruct((E, PW), jnp.float32),
        ),
        mesh=_mesh,
        compiler_params=_untiled,
        scratch_types=[pltpu.SemaphoreType.DMA],
    )
    def gather_kernel(p_hbm, pn_hbm, si_hbm, di_hbm, gp_hbm, gq_hbm, sem):
        def body(s_vmem, d_vmem, op_vmem, oq_vmem):
            copies = []
            for k in range(KGP):
                sl = pl.ds(k * GW, GW)
                copies.append(pltpu.async_copy(p_hbm.at[s_vmem.at[0, sl]],
                                               op_vmem.at[sl], sem))
                copies.append(pltpu.async_copy(pn_hbm.at[d_vmem.at[0, sl]],
                                               oq_vmem.at[sl], sem))
            for c in copies:
                c.wait()

        pltpu.emit_pipeline(
            body,
            grid=(CHUNKS // KGP,),
            in_specs=[
                pl.BlockSpec((1, KGP * GW), lambda i: (0, i)),
                pl.BlockSpec((1, KGP * GW), lambda i: (0, i)),
            ],
            out_specs=[
                pl.BlockSpec((KGP * GW, PW), lambda i: (i, 0)),
                pl.BlockSpec((KGP * GW, PW), lambda i: (i, 0)),
            ],
            core_axis_name=("core", "subcore"),
            dimension_semantics=(pltpu.PARALLEL,),
        )(si_hbm, di_hbm, gp_hbm, gq_hbm)

    return gather_kernel(P, PN, src_row, dst_row)


def _edge_body(ga_ref, gb_ref, gp_ref, gq_ref, ea_ref, WaeT_ref, wd_ref,
               WqT_ref, bq_ref, Wc2T_ref, bc2_ref, r_ref, u_ref):
    xsum = ga_ref[...] + gb_ref[...]
    pp = gp_ref[...] + gq_ref[...]
    dp = pp[:, 0:3]
    dq = pp[:, 3:6]
    d_sq = jnp.sum(dp * dp, axis=1, keepdims=True)
    d2 = jnp.sum(dq * dq, axis=1, keepdims=True)
    rinv = lax.rsqrt(d2)
    pre = xsum + d_sq * wd_ref[...] + lax.dot_general(
        ea_ref[...], WaeT_ref[...], (((1,), (0,)), ((), ())),
        preferred_element_type=jnp.float32)
    r = jnp.maximum(pre, 0.0)
    t = jnp.maximum(
        lax.dot_general(r, WqT_ref[...], (((1,), (0,)), ((), ())),
                        preferred_element_type=jnp.float32) + bq_ref[...], 0.0)
    coord = lax.dot_general(t, Wc2T_ref[...], (((1,), (0,)), ((), ())),
                            preferred_element_type=jnp.float32) + bc2_ref[...]
    pij = dq * coord * rinv
    ones = jnp.ones((r.shape[0], 1), jnp.float32)
    pad = jnp.zeros((r.shape[0], PW - 4), jnp.float32)
    r_ref[...] = r
    u_ref[...] = jnp.concatenate([pij, ones, pad], axis=1)


def _edge_compute(ga, gb, gp, gq, ea, WaeT, wd, WqT, bq, Wc2T, bc2):
    return pl.pallas_call(
        _edge_body,
        grid=(E // EBLK,),
        in_specs=[
            pl.BlockSpec((EBLK, D), lambda i: (i, 0)),
            pl.BlockSpec((EBLK, D), lambda i: (i, 0)),
            pl.BlockSpec((EBLK, PW), lambda i: (i, 0)),
            pl.BlockSpec((EBLK, PW), lambda i: (i, 0)),
            pl.BlockSpec((EBLK, 4), lambda i: (i, 0)),
            pl.BlockSpec((4, D), lambda i: (0, 0)),
            pl.BlockSpec((1, D), lambda i: (0, 0)),
            pl.BlockSpec((D, D), lambda i: (0, 0)),
            pl.BlockSpec((1, D), lambda i: (0, 0)),
            pl.BlockSpec((D, 3), lambda i: (0, 0)),
            pl.BlockSpec((1, 3), lambda i: (0, 0)),
        ],
        out_specs=[
            pl.BlockSpec((EBLK, D), lambda i: (i, 0)),
            pl.BlockSpec((EBLK, PW), lambda i: (i, 0)),
        ],
        out_shape=[
            jax.ShapeDtypeStruct((E, D), jnp.float32),
            jax.ShapeDtypeStruct((E, PW), jnp.float32),
        ],
    )(ga, gb, gp, gq, ea, WaeT, wd, WqT, bq, Wc2T, bc2)


KS1 = 2
KS2 = 10


def _scatter_main(v, dst2d, zrows):
    @functools.partial(
        pl.kernel,
        out_type=jax.ShapeDtypeStruct((NC, NP, D), jnp.float32),
        mesh=_mesh,
        scratch_types=[
            pltpu.VMEM((KS1 * GW, D), jnp.float32),
            pltpu.VMEM((KS1, GW), jnp.int32),
            pltpu.VMEM_SHARED((NP, D), jnp.float32),
        ],
    )
    def scatter_kernel(v_hbm, di_hbm, z_hbm, o_hbm, rv, iv, acc_sh):
        cid = lax.axis_index("core")
        sid = lax.axis_index("subcore")
        wid = sid * NC + cid
        pltpu.sync_copy(z_hbm, acc_sh.at[pl.ds(sid * ROWS_PER_TILE, ROWS_PER_TILE)])
        plsc.subcore_barrier()

        @pl.loop(wid, CHUNKS // KS1, step=NW)
        def _(c):
            pltpu.sync_copy(di_hbm.at[pl.ds(c * KS1, KS1)], iv)
            pltpu.sync_copy(v_hbm.at[pl.ds(c * (KS1 * GW), KS1 * GW)], rv)
            for k in range(KS1):
                pltpu.sync_copy(rv.at[pl.ds(k * GW, GW)], acc_sh.at[iv.at[k]],
                                add=True)

        plsc.subcore_barrier()
        pltpu.sync_copy(
            acc_sh.at[pl.ds(sid * ROWS_PER_TILE, ROWS_PER_TILE)],
            o_hbm.at[cid, pl.ds(sid * ROWS_PER_TILE, ROWS_PER_TILE)],
        )

    return scatter_kernel(v, dst2d, zrows)


def _scatter_pos(u, dst2d, zrows):
    @functools.partial(
        pl.kernel,
        out_type=jax.ShapeDtypeStruct((NC, NP, PW), jnp.float32),
        mesh=_mesh,
        compiler_params=_untiled,
        scratch_types=[
            pltpu.VMEM((KS2 * GW, PW), jnp.float32),
            pltpu.VMEM((KS2, GW), jnp.int32),
            pltpu.VMEM_SHARED((NP, PW), jnp.float32),
        ],
    )
    def scatter_kernel(u_hbm, di_hbm, z_hbm, o_hbm, uv, iv, acc_sh):
        cid = lax.axis_index("core")
        sid = lax.axis_index("subcore")
        wid = sid * NC + cid
        pltpu.sync_copy(z_hbm, acc_sh.at[pl.ds(sid * ROWS_PER_TILE, ROWS_PER_TILE)])
        plsc.subcore_barrier()

        @pl.loop(wid, CHUNKS // KS2, step=NW)
        def _(c):
            pltpu.sync_copy(di_hbm.at[pl.ds(c * KS2, KS2)], iv)
            pltpu.sync_copy(u_hbm.at[pl.ds(c * (KS2 * GW), KS2 * GW)], uv)
            for k in range(KS2):
                pltpu.sync_copy(uv.at[pl.ds(k * GW, GW)], acc_sh.at[iv.at[k]],
                                add=True)

        plsc.subcore_barrier()
        pltpu.sync_copy(
            acc_sh.at[pl.ds(sid * ROWS_PER_TILE, ROWS_PER_TILE)],
            o_hbm.at[cid, pl.ds(sid * ROWS_PER_TILE, ROWS_PER_TILE)],
        )

    return scatter_kernel(u, dst2d, zrows)


def _node_body(p0_ref, p1_ref, u0_ref, u1_ref, x_ref, Wm2T_ref, bm2_ref,
               Wn1aT_ref, Wn1bT_ref, bn1_ref, Wn2T_ref, bn2_ref, h_ref, pu_ref):
    S = p0_ref[...] + p1_ref[...]
    su = u0_ref[...] + u1_ref[...]
    pupd = su[:, 0:3]
    cnt = su[:, 3:4]
    msum = lax.dot_general(S, Wm2T_ref[...], (((1,), (0,)), ((), ())),
                           preferred_element_type=jnp.float32) + cnt * bm2_ref[...]
    m_i = msum / jnp.maximum(cnt, 1.0)
    h1 = jnp.maximum(
        lax.dot_general(x_ref[...], Wn1aT_ref[...], (((1,), (0,)), ((), ())),
                        preferred_element_type=jnp.float32)
        + lax.dot_general(m_i, Wn1bT_ref[...], (((1,), (0,)), ((), ())),
                          preferred_element_type=jnp.float32)
        + bn1_ref[...], 0.0)
    h_ref[...] = lax.dot_general(h1, Wn2T_ref[...], (((1,), (0,)), ((), ())),
                                 preferred_element_type=jnp.float32) + bn2_ref[...]
    pu_ref[...] = pupd


def _node_mlp(p0, p1, u0, u1, x, Wm2T, bm2, Wn1aT, Wn1bT, bn1, Wn2T, bn2):
    return pl.pallas_call(
        _node_body,
        grid=(N // NBLK,),
        in_specs=[
            pl.BlockSpec((NBLK, D), lambda i: (i, 0)),
            pl.BlockSpec((NBLK, D), lambda i: (i, 0)),
            pl.BlockSpec((NBLK, PW), lambda i: (i, 0)),
            pl.BlockSpec((NBLK, PW), lambda i: (i, 0)),
            pl.BlockSpec((NBLK, D), lambda i: (i, 0)),
            pl.BlockSpec((D, D), lambda i: (0, 0)),
            pl.BlockSpec((1, D), lambda i: (0, 0)),
            pl.BlockSpec((D, D), lambda i: (0, 0)),
            pl.BlockSpec((D, D), lambda i: (0, 0)),
            pl.BlockSpec((1, D), lambda i: (0, 0)),
            pl.BlockSpec((D, D), lambda i: (0, 0)),
            pl.BlockSpec((1, D), lambda i: (0, 0)),
        ],
        out_specs=[
            pl.BlockSpec((NBLK, D), lambda i: (i, 0)),
            pl.BlockSpec((NBLK, 3), lambda i: (i, 0)),
        ],
        out_shape=[
            jax.ShapeDtypeStruct((N, D), jnp.float32),
            jax.ShapeDtypeStruct((N, 3), jnp.float32),
        ],
    )(p0, p1, u0, u1, x, Wm2T, bm2, Wn1aT, Wn1bT, bn1, Wn2T, bn2)


def kernel(x, edge_index, pos, pos_init, edge_attr, Wm1, bm1, Wm2, bm2,
           Wn1, bn1, Wn2, bn2, Wc1, bc1, Wc2, bc2):
    src = edge_index[0].astype(jnp.int32)
    dst = edge_index[1].astype(jnp.int32)

    A = Wm1[:, :D]
    B = Wm1[:, D:2 * D]
    wd = Wm1[:, 2 * D].reshape(1, D)
    WaeT = Wm1[:, 2 * D + 1:].T
    Wq = jnp.dot(Wc1, Wm2, precision="highest")
    bq = (jnp.dot(Wc1, bm2, precision="highest") + bc1).reshape(1, D)

    P = jnp.concatenate([pos, pos_init, jnp.zeros((N, PW - 6), jnp.float32)], axis=1)

    ta, tb = _build_tables(x, A, B, bm1.reshape(1, D))
    src_row = src.reshape(1, E)
    dst_row = dst.reshape(1, E)
    ga, gb = _gather_main(ta, tb, src_row, dst_row)
    gp, gq = _gather_pos(P, -P, src_row, dst_row)
    r, u = _edge_compute(ga, gb, gp, gq, edge_attr, WaeT, wd, Wq.T, bq,
                         Wc2.T, bc2.reshape(1, 3))
    dst2d = dst.reshape(CHUNKS, GW)
    partials = _scatter_main(r, dst2d, jnp.zeros((ROWS_PER_TILE, D), jnp.float32))
    upart = _scatter_pos(u, dst2d, jnp.zeros((ROWS_PER_TILE, PW), jnp.float32))
    h, pos_upd = _node_mlp(partials[0], partials[1], upart[0], upart[1], x,
                           Wm2.T, bm2.reshape(1, D), Wn1[:, :D].T, Wn1[:, D:].T,
                           bn1.reshape(1, D), Wn2.T, bn2.reshape(1, D))
    return (h, pos_upd)
